# Initial kernel scaffold; baseline (speedup 1.0000x reference)
import jax
import jax.numpy as jnp
from jax.experimental import pallas as pl


def kernel(x, W_in, conv_w, conv_b, W_xproj, W_dt, b_dt, A_log, D, W_out):
    raise NotImplementedError("write your pallas kernel here")


# speedup vs baseline: 21.1456x; 21.1456x over previous
from functools import partial

import jax
import jax.numpy as jnp
from jax.experimental import pallas as pl
from jax.experimental.pallas import tpu as pltpu

_F32 = jnp.float32


def _silu(v):
    return v * jax.nn.sigmoid(v)


def _front_kernel(x_ref, Wu_ref, Wx_ref, Wdt_ref, cw_ref, cb_ref, bdt_ref,
                  u_ref, dt_ref, BT_ref, CT_ref, carry_ref,
                  *, dt_rank, n_state, d_conv):
    j = pl.program_id(1)

    @pl.when(j == 0)
    def _():
        carry_ref[...] = jnp.zeros_like(carry_ref)

    x = x_ref[0]
    u_raw = jnp.dot(x, Wu_ref[...], preferred_element_type=_F32)

    Lt = u_raw.shape[0]
    full = jnp.concatenate([carry_ref[...], u_raw], axis=0)
    carry_ref[...] = full[Lt:Lt + 8]
    acc = jnp.zeros_like(u_raw) + cb_ref[...]
    for k in range(d_conv):
        off = 8 - (d_conv - 1) + k
        acc = acc + full[off:off + Lt] * cw_ref[k:k + 1, :]
    u = _silu(acc)
    u_ref[...] = u

    xdbl = jnp.dot(u, Wx_ref[...], preferred_element_type=_F32)
    dt_low = xdbl[:, :dt_rank]
    dtv = jax.nn.softplus(
        jnp.dot(dt_low, Wdt_ref[...], preferred_element_type=_F32) + bdt_ref[...])
    dt_ref[...] = dtv
    BT_ref[...] = xdbl[:, dt_rank:dt_rank + n_state].T
    CT_ref[...] = xdbl[:, dt_rank + n_state:dt_rank + 2 * n_state].T


def _scan_kernel(u_ref, dt_ref, BT_ref, CT_ref, A_ref, D_ref,
                 y_ref, h_ref):
    j = pl.program_id(1)

    @pl.when(j == 0)
    def _():
        h_ref[...] = jnp.zeros_like(h_ref)

    u = u_ref[...]
    dt = dt_ref[...]
    A = A_ref[...]
    dtu = dt * u
    BT = BT_ref[...]
    CT = CT_ref[...]
    h = h_ref[...]
    Tc = u.shape[0]
    for t in range(Tc):
        dA = jnp.exp(dt[t:t + 1, :] * A)
        h = dA * h + dtu[t:t + 1, :] * BT[:, t:t + 1]
        y_ref[t:t + 1, :] = jnp.sum(h * CT[:, t:t + 1], axis=0, keepdims=True)
    h_ref[...] = h
    y_ref[...] = y_ref[...] + u * D_ref[...]


def _out_kernel(x_ref, Wr_ref, y_ref, Wo_ref, o_ref):
    res = jnp.dot(x_ref[0], Wr_ref[...], preferred_element_type=_F32)
    g = y_ref[...] * _silu(res)
    o_ref[0] = jnp.dot(g, Wo_ref[...], preferred_element_type=_F32)


def kernel(x, W_in, conv_w, conv_b, W_xproj, W_dt, b_dt, A_log, D, W_out):
    B, L, d_model = x.shape
    d_inner, d_conv = conv_w.shape
    dt_rank = W_dt.shape[0]
    n_state = A_log.shape[1]

    Lt = min(512, L)
    J1 = L // Lt
    Tc = min(128, L)
    J2 = L // Tc
    dblk = min(512, d_inner)
    Gd = d_inner // dblk

    Wu = W_in[:, :d_inner]
    Wr = W_in[:, d_inner:]
    cwT = conv_w.T
    cb = conv_b[None, :]
    bdt = b_dt[None, :]
    AT = (-jnp.exp(A_log)).T
    Dc = jnp.concatenate([D] * B)[None, :]

    cp = pltpu.CompilerParams(
        dimension_semantics=("parallel", "arbitrary"),
        vmem_limit_bytes=56 * 1024 * 1024,
    )

    u_c, dt_c, BT, CT = pl.pallas_call(
        partial(_front_kernel, dt_rank=dt_rank, n_state=n_state, d_conv=d_conv),
        grid=(B, J1),
        in_specs=[
            pl.BlockSpec((1, Lt, d_model), lambda b, j: (b, j, 0)),
            pl.BlockSpec((d_model, d_inner), lambda b, j: (0, 0)),
            pl.BlockSpec((d_inner, dt_rank + 2 * n_state), lambda b, j: (0, 0)),
            pl.BlockSpec((dt_rank, d_inner), lambda b, j: (0, 0)),
            pl.BlockSpec((d_conv, d_inner), lambda b, j: (0, 0)),
            pl.BlockSpec((1, d_inner), lambda b, j: (0, 0)),
            pl.BlockSpec((1, d_inner), lambda b, j: (0, 0)),
        ],
        out_specs=[
            pl.BlockSpec((Lt, d_inner), lambda b, j: (j, b)),
            pl.BlockSpec((Lt, d_inner), lambda b, j: (j, b)),
            pl.BlockSpec((n_state, Lt), lambda b, j: (b, j)),
            pl.BlockSpec((n_state, Lt), lambda b, j: (b, j)),
        ],
        out_shape=[
            jax.ShapeDtypeStruct((L, B * d_inner), _F32),
            jax.ShapeDtypeStruct((L, B * d_inner), _F32),
            jax.ShapeDtypeStruct((B * n_state, L), _F32),
            jax.ShapeDtypeStruct((B * n_state, L), _F32),
        ],
        scratch_shapes=[pltpu.VMEM((8, d_inner), _F32)],
        compiler_params=cp,
        name="mamba_front",
    )(x, Wu, W_xproj, W_dt, cwT, cb, bdt)

    y2 = pl.pallas_call(
        _scan_kernel,
        grid=(B * Gd, J2),
        in_specs=[
            pl.BlockSpec((Tc, dblk), lambda g, j: (j, g)),
            pl.BlockSpec((Tc, dblk), lambda g, j: (j, g)),
            pl.BlockSpec((n_state, Tc), lambda g, j: (g // Gd, j)),
            pl.BlockSpec((n_state, Tc), lambda g, j: (g // Gd, j)),
            pl.BlockSpec((n_state, dblk), lambda g, j: (0, g % Gd)),
            pl.BlockSpec((1, dblk), lambda g, j: (0, g)),
        ],
        out_specs=pl.BlockSpec((Tc, dblk), lambda g, j: (j, g)),
        out_shape=jax.ShapeDtypeStruct((L, B * d_inner), _F32),
        scratch_shapes=[pltpu.VMEM((n_state, dblk), _F32)],
        compiler_params=cp,
        name="mamba_scan",
    )(u_c, dt_c, BT, CT, AT, Dc)

    o = pl.pallas_call(
        _out_kernel,
        grid=(B, J1),
        in_specs=[
            pl.BlockSpec((1, Lt, d_model), lambda b, j: (b, j, 0)),
            pl.BlockSpec((d_model, d_inner), lambda b, j: (0, 0)),
            pl.BlockSpec((Lt, d_inner), lambda b, j: (j, b)),
            pl.BlockSpec((d_inner, d_model), lambda b, j: (0, 0)),
        ],
        out_specs=pl.BlockSpec((1, Lt, d_model), lambda b, j: (b, j, 0)),
        out_shape=jax.ShapeDtypeStruct((B, L, d_model), _F32),
        compiler_params=cp,
        name="mamba_out",
    )(x, Wr, y2, W_out)

    return o
